# baseline (speedup 1.0000x reference)
import jax
import jax.numpy as jnp
from jax import lax
from jax.experimental import pallas as pl
from jax.experimental.pallas import tpu as pltpu
from jax.experimental.pallas import tpu_sc as plsc

BATCH = 16384
EMB = 64
NROWS = 1000000
NC = 2
NS = 16
NW = NC * NS
B_PER = BATCH // NW
CW = 512
CSH = 9
TAIL0 = (NROWS // 128) * 128
LASTC0 = TAIL0 - CW
NSLOT = 16


def _iota16():
    return lax.broadcasted_iota(jnp.int32, (16,), 0)


def _extract(vec, lane):
    return jnp.sum(jnp.where(_iota16() == lane, vec, 0))


def _gather_body(su_hbm, bu_hbm, sv_hbm, bv_hbm, uembT, iembT,
                 utail_hbm, itail_hbm, urows_hbm, vrows_hbm,
                 sidx, bidx, bufs, tailbuf, stage, fsem, wsem):
    wid = lax.axis_index("s") * NC + lax.axis_index("c")
    base = wid * B_PER
    lane16 = _iota16()

    def run_side(si_hbm, bi_hbm, tabT, tail_hbm, rows_hbm):
        pltpu.sync_copy(si_hbm.at[pl.ds(base, B_PER)], sidx)
        pltpu.sync_copy(bi_hbm.at[pl.ds(base, B_PER)], bidx)
        pltpu.sync_copy(tail_hbm, tailbuf)

        first = _extract(sidx[pl.ds(0, 16)], 0)
        last = _extract(sidx[pl.ds(B_PER - 16, 16)], 15)
        cb = (first >> 7) << 7
        last_cl = jnp.minimum(last, TAIL0 - 1)
        nchunks = jnp.maximum(((last_cl - cb) >> CSH) + 1, 0)

        def fire(k):
            c0 = pl.multiple_of(jnp.minimum(cb + k * CW, LASTC0), 128)
            pltpu.async_copy(tabT.at[:, pl.ds(c0, CW)],
                             bufs.at[k % 2], fsem.at[k % 2])

        def wait_chunk(k):
            pltpu.make_async_copy(tabT.at[:, pl.ds(0, CW)],
                                  bufs.at[k % 2], fsem.at[k % 2]).wait()

        def prep(p):
            pe = jnp.minimum(p, B_PER - 1)
            p16 = (pe >> 4) << 4
            ln = pe & 15
            sch = sidx[pl.ds(p16, 16)]
            bch = bidx[pl.ds(p16, 16)]
            return _extract(sch, ln), _extract(bch, ln)

        def emit(pos, si_s, bi_s, vals4):
            slot = pos & (NSLOT - 1)
            for j in range(4):
                stage[slot, pl.ds(16 * j, 16)] = vals4[j]

            @pl.when(pos >= NSLOT)
            def _():
                pltpu.make_async_copy(rows_hbm.at[0], stage.at[0],
                                      wsem.at[slot]).wait()
            pltpu.async_copy(stage.at[slot], rows_hbm.at[bi_s],
                             wsem.at[slot])

        @pl.when(nchunks > 0)
        def _():
            fire(0)

        def chunk_body(k, pos):
            @pl.when(k + 1 < nchunks)
            def _():
                fire(k + 1)
            wait_chunk(k)
            c0 = jnp.minimum(cb + k * CW, LASTC0)
            par16 = jnp.full((16,), k % 2, jnp.int32)

            si0, bi0 = prep(pos)
            cont0 = ((pos < B_PER) & (((si0 - cb) >> CSH) == k)
                     & (si0 < TAIL0))

            def wbody(carry):
                p, _, si_s, bi_s = carry
                col16 = jnp.full((16,), si_s - c0, jnp.int32)
                vals = []
                for j in range(4):
                    d16 = lane16 + 16 * j
                    vals.append(plsc.load_gather(bufs, [par16, d16, col16]))
                emit(p, si_s, bi_s, vals)
                np_ = p + 1
                si_n, bi_n = prep(np_)
                cont = ((np_ < B_PER) & (((si_n - cb) >> CSH) == k)
                        & (si_n < TAIL0))
                return (np_, cont, si_n, bi_n)

            out = lax.while_loop(lambda c: c[1], wbody,
                                 (pos, cont0, si0, bi0))
            return out[0]

        pos = lax.fori_loop(0, nchunks, chunk_body, 0)

        si0, bi0 = prep(pos)

        def tbody(carry):
            p, _, si_s, bi_s = carry
            col16 = jnp.full((16,), si_s - TAIL0, jnp.int32)
            vals = []
            for j in range(4):
                d16 = lane16 + 16 * j
                vals.append(plsc.load_gather(tailbuf, [d16, col16]))
            emit(p, si_s, bi_s, vals)
            np_ = p + 1
            si_n, bi_n = prep(np_)
            return (np_, np_ < B_PER, si_n, bi_n)

        lax.while_loop(lambda c: c[1], tbody,
                       (pos, pos < B_PER, si0, bi0))

        for s in range(NSLOT):
            pltpu.make_async_copy(rows_hbm.at[0], stage.at[0],
                                  wsem.at[s]).wait()

    run_side(su_hbm, bu_hbm, uembT, utail_hbm, urows_hbm)
    run_side(sv_hbm, bv_hbm, iembT, itail_hbm, vrows_hbm)


DCH = 128


def _dot_body(urows_hbm, vrows_hbm, out_hbm, ubuf, vbuf, outv, sem):
    wid = lax.axis_index("s") * NC + lax.axis_index("c")
    base = wid * B_PER
    lane = _iota16()
    nch = B_PER // DCH

    def fire(c):
        sl = pl.ds(base + c * DCH, DCH)
        p = c % 2
        pltpu.async_copy(urows_hbm.at[sl], ubuf.at[p], sem.at[p])
        pltpu.async_copy(vrows_hbm.at[sl], vbuf.at[p], sem.at[p])

    def wait(c):
        p = c % 2
        pltpu.make_async_copy(urows_hbm.at[pl.ds(0, DCH)], ubuf.at[p],
                              sem.at[p]).wait()
        pltpu.make_async_copy(vrows_hbm.at[pl.ds(0, DCH)], vbuf.at[p],
                              sem.at[p]).wait()

    fire(0)
    for c in range(nch):
        if c + 1 < nch:
            fire(c + 1)
        wait(c)
        p = c % 2

        def group(g, _):
            row16 = g * 16 + lane
            par = jnp.full((16,), p, jnp.int32)
            acc = jnp.zeros((16,), jnp.float32)
            for d in range(EMB):
                col = jnp.full((16,), d, jnp.int32)
                eu = plsc.load_gather(ubuf, [par, row16, col])
                ev = plsc.load_gather(vbuf, [par, row16, col])
                acc += eu * ev
            outv[pl.ds(c * DCH + g * 16, 16)] = acc
            return 0

        lax.fori_loop(0, DCH // 16, group, 0)

    pltpu.sync_copy(outv, out_hbm.at[pl.ds(base, B_PER)])


@jax.jit
def kernel(u, v, user_emb, item_emb):
    bu = jnp.argsort(u).astype(jnp.int32)
    su = jnp.take(u, bu)
    bv = jnp.argsort(v).astype(jnp.int32)
    sv = jnp.take(v, bv)
    uembT = user_emb.T
    iembT = item_emb.T
    utail = user_emb[TAIL0:].T
    itail = item_emb[TAIL0:].T

    mesh = plsc.VectorSubcoreMesh(core_axis_name="c", subcore_axis_name="s")
    cp = pltpu.CompilerParams(needs_layout_passes=False)

    gather_k = pl.kernel(
        _gather_body,
        out_type=(pltpu.HBM((BATCH, EMB), jnp.float32),
                  pltpu.HBM((BATCH, EMB), jnp.float32)),
        mesh=mesh,
        compiler_params=cp,
        scratch_types=[
            pltpu.VMEM((B_PER,), jnp.int32),
            pltpu.VMEM((B_PER,), jnp.int32),
            pltpu.VMEM((2, EMB, CW), jnp.float32),
            pltpu.VMEM((EMB, NROWS - TAIL0), jnp.float32),
            pltpu.VMEM((NSLOT, EMB), jnp.float32),
            pltpu.SemaphoreType.DMA((2,)),
            pltpu.SemaphoreType.DMA((NSLOT,)),
        ],
    )
    urows, vrows = gather_k(su, bu, sv, bv, uembT, iembT, utail, itail)

    dot_k = pl.kernel(
        _dot_body,
        out_type=jax.ShapeDtypeStruct((BATCH,), jnp.float32),
        mesh=mesh,
        compiler_params=cp,
        scratch_types=[
            pltpu.VMEM((2, DCH, EMB), jnp.float32),
            pltpu.VMEM((2, DCH, EMB), jnp.float32),
            pltpu.VMEM((B_PER,), jnp.float32),
            pltpu.SemaphoreType.DMA((2,)),
        ],
    )
    return dot_k(urows, vrows)


# speedup vs baseline: 1.0338x; 1.0338x over previous
import jax
import jax.numpy as jnp
from jax import lax
from jax.experimental import pallas as pl
from jax.experimental.pallas import tpu as pltpu
from jax.experimental.pallas import tpu_sc as plsc

BATCH = 16384
EMB = 64
NROWS = 1000000
NC = 2
NS = 16
NW = NC * NS
B_PER = BATCH // NW
CW = 512
CSH = 9
TAIL0 = (NROWS // 128) * 128
LASTC0 = TAIL0 - CW
NSLOT = 16


def _iota16():
    return lax.broadcasted_iota(jnp.int32, (16,), 0)


def _extract(vec, lane):
    return jnp.sum(jnp.where(_iota16() == lane, vec, 0))


def _gather_body(su_hbm, bu_hbm, sv_hbm, bv_hbm, uembT, iembT,
                 utail_hbm, itail_hbm, urows_hbm, vrows_hbm,
                 sidx, bidx, bufs, tailbuf, stage, fsem, wsem):
    wid = lax.axis_index("s") * NC + lax.axis_index("c")
    base = wid * B_PER
    lane16 = _iota16()

    def run_side(si_hbm, bi_hbm, tabT, tail_hbm, rows_hbm):
        pltpu.sync_copy(si_hbm.at[pl.ds(base, B_PER)], sidx)
        pltpu.sync_copy(bi_hbm.at[pl.ds(base, B_PER)], bidx)
        pltpu.sync_copy(tail_hbm, tailbuf)

        first = _extract(sidx[pl.ds(0, 16)], 0)
        last = _extract(sidx[pl.ds(B_PER - 16, 16)], 15)
        cb = (first >> 7) << 7
        last_cl = jnp.minimum(last, TAIL0 - 1)
        nchunks = jnp.maximum(((last_cl - cb) >> CSH) + 1, 0)

        def fire(k):
            c0 = pl.multiple_of(jnp.minimum(cb + k * CW, LASTC0), 128)
            pltpu.async_copy(tabT.at[:, pl.ds(c0, CW)],
                             bufs.at[k % 2], fsem.at[k % 2])

        def wait_chunk(k):
            pltpu.make_async_copy(tabT.at[:, pl.ds(0, CW)],
                                  bufs.at[k % 2], fsem.at[k % 2]).wait()

        def prep(p):
            pe = jnp.minimum(p, B_PER - 1)
            p16 = (pe >> 4) << 4
            ln = pe & 15
            sch = sidx[pl.ds(p16, 16)]
            bch = bidx[pl.ds(p16, 16)]
            return _extract(sch, ln), _extract(bch, ln)

        def emit(pos, si_s, bi_s, vals4):
            slot = pos & (NSLOT - 1)
            for j in range(4):
                stage[slot, pl.ds(16 * j, 16)] = vals4[j]

            @pl.when(pos >= NSLOT)
            def _():
                pltpu.make_async_copy(rows_hbm.at[0], stage.at[0],
                                      wsem.at[slot]).wait()
            pltpu.async_copy(stage.at[slot], rows_hbm.at[bi_s],
                             wsem.at[slot])

        @pl.when(nchunks > 0)
        def _():
            fire(0)

        def chunk_body(k, pos):
            @pl.when(k + 1 < nchunks)
            def _():
                fire(k + 1)
            wait_chunk(k)
            c0 = jnp.minimum(cb + k * CW, LASTC0)
            par16 = jnp.full((16,), k % 2, jnp.int32)

            si0, bi0 = prep(pos)
            cont0 = ((pos < B_PER) & (((si0 - cb) >> CSH) == k)
                     & (si0 < TAIL0))

            def wbody(carry):
                p, _, si_s, bi_s = carry
                col16 = jnp.full((16,), si_s - c0, jnp.int32)
                vals = []
                for j in range(4):
                    d16 = lane16 + 16 * j
                    vals.append(plsc.load_gather(bufs, [par16, d16, col16]))
                emit(p, si_s, bi_s, vals)
                np_ = p + 1
                si_n, bi_n = prep(np_)
                cont = ((np_ < B_PER) & (((si_n - cb) >> CSH) == k)
                        & (si_n < TAIL0))
                return (np_, cont, si_n, bi_n)

            out = lax.while_loop(lambda c: c[1], wbody,
                                 (pos, cont0, si0, bi0))
            return out[0]

        pos = lax.fori_loop(0, nchunks, chunk_body, 0)

        si0, bi0 = prep(pos)

        def tbody(carry):
            p, _, si_s, bi_s = carry
            col16 = jnp.full((16,), si_s - TAIL0, jnp.int32)
            vals = []
            for j in range(4):
                d16 = lane16 + 16 * j
                vals.append(plsc.load_gather(tailbuf, [d16, col16]))
            emit(p, si_s, bi_s, vals)
            np_ = p + 1
            si_n, bi_n = prep(np_)
            return (np_, np_ < B_PER, si_n, bi_n)

        lax.while_loop(lambda c: c[1], tbody,
                       (pos, pos < B_PER, si0, bi0))

        for s in range(NSLOT):
            pltpu.make_async_copy(rows_hbm.at[0], stage.at[0],
                                  wsem.at[s]).wait()

    run_side(su_hbm, bu_hbm, uembT, utail_hbm, urows_hbm)
    run_side(sv_hbm, bv_hbm, iembT, itail_hbm, vrows_hbm)


DCH = 128


def _dot_body(urows_hbm, vrows_hbm, out_hbm, ubuf, vbuf, outv, sem):
    wid = lax.axis_index("s") * NC + lax.axis_index("c")
    base = wid * B_PER
    lane = _iota16()
    nch = B_PER // DCH

    def fire(c):
        sl = pl.ds(base + c * DCH, DCH)
        p = c % 2
        pltpu.async_copy(urows_hbm.at[sl], ubuf.at[p], sem.at[p])
        pltpu.async_copy(vrows_hbm.at[sl], vbuf.at[p], sem.at[p])

    def wait(c):
        p = c % 2
        pltpu.make_async_copy(urows_hbm.at[pl.ds(0, DCH)], ubuf.at[p],
                              sem.at[p]).wait()
        pltpu.make_async_copy(vrows_hbm.at[pl.ds(0, DCH)], vbuf.at[p],
                              sem.at[p]).wait()

    fire(0)
    for c in range(nch):
        if c + 1 < nch:
            fire(c + 1)
        wait(c)
        p = c % 2

        def group(g, _):
            row16 = g * 16 + lane
            par = jnp.full((16,), p, jnp.int32)
            acc = jnp.zeros((16,), jnp.float32)
            for d in range(EMB):
                col = jnp.full((16,), d, jnp.int32)
                eu = plsc.load_gather(ubuf, [par, row16, col])
                ev = plsc.load_gather(vbuf, [par, row16, col])
                acc += eu * ev
            outv[pl.ds(c * DCH + g * 16, 16)] = acc
            return 0

        lax.fori_loop(0, DCH // 16, group, 0)

    pltpu.sync_copy(outv, out_hbm.at[pl.ds(base, B_PER)])


@jax.jit
def kernel(u, v, user_emb, item_emb):
    pos = lax.broadcasted_iota(jnp.int32, (BATCH,), 0)
    su, bu = lax.sort((u, pos), num_keys=1)
    sv, bv = lax.sort((v, pos), num_keys=1)
    uembT = user_emb.T
    iembT = item_emb.T
    utail = user_emb[TAIL0:].T
    itail = item_emb[TAIL0:].T

    mesh = plsc.VectorSubcoreMesh(core_axis_name="c", subcore_axis_name="s")
    cp = pltpu.CompilerParams(needs_layout_passes=False)

    gather_k = pl.kernel(
        _gather_body,
        out_type=(pltpu.HBM((BATCH, EMB), jnp.float32),
                  pltpu.HBM((BATCH, EMB), jnp.float32)),
        mesh=mesh,
        compiler_params=cp,
        scratch_types=[
            pltpu.VMEM((B_PER,), jnp.int32),
            pltpu.VMEM((B_PER,), jnp.int32),
            pltpu.VMEM((2, EMB, CW), jnp.float32),
            pltpu.VMEM((EMB, NROWS - TAIL0), jnp.float32),
            pltpu.VMEM((NSLOT, EMB), jnp.float32),
            pltpu.SemaphoreType.DMA((2,)),
            pltpu.SemaphoreType.DMA((NSLOT,)),
        ],
    )
    urows, vrows = gather_k(su, bu, sv, bv, uembT, iembT, utail, itail)

    dot_k = pl.kernel(
        _dot_body,
        out_type=jax.ShapeDtypeStruct((BATCH,), jnp.float32),
        mesh=mesh,
        compiler_params=cp,
        scratch_types=[
            pltpu.VMEM((2, DCH, EMB), jnp.float32),
            pltpu.VMEM((2, DCH, EMB), jnp.float32),
            pltpu.VMEM((B_PER,), jnp.float32),
            pltpu.SemaphoreType.DMA((2,)),
        ],
    )
    return dot_k(urows, vrows)
